# baseline (speedup 1.0000x reference)
import jax
import jax.numpy as jnp
from jax.experimental import pallas as pl
from jax.experimental.pallas import tpu as pltpu

_N1 = 8192
_N2 = 8192
_C = 16
_R = 256
_NBLK = _N1 // _R


def _vsum(x):
    return jnp.sum(x.reshape(_R // 8, 8, _N2), axis=0)


def _body(y2r_ref, y1r_ref, f2_ref, f_ref, y2c_ref, y1c_ref, out_ref, acc_ref):
    i = pl.program_id(0)
    f2b = f2_ref[...]
    fb = f_ref[...]
    cols = y2r_ref[...]
    mask22 = y2c_ref[...] == cols
    mask12 = y1c_ref[...] == cols
    s_m1 = _vsum(jnp.where(mask22, f2b, 0.0))
    s_t1 = _vsum(f2b)
    s_m2 = _vsum(jnp.where(mask12, fb, 0.0))
    s_t2 = _vsum(fb)

    @pl.when(i == 0)
    def _init():
        acc_ref[0] = s_m1
        acc_ref[1] = s_t1
        acc_ref[2] = s_m2
        acc_ref[3] = s_t2

    @pl.when(i > 0)
    def _accum():
        acc_ref[0] += s_m1
        acc_ref[1] += s_t1
        acc_ref[2] += s_m2
        acc_ref[3] += s_t2

    @pl.when(i == _NBLK - 1)
    def _finalize():
        y2row = y2r_ref[...]
        y1row = y1r_ref[...]
        c22 = jnp.float32(0.0)
        c12 = jnp.float32(0.0)
        for c in range(_C):
            n2c = jnp.sum((y2row == c).astype(jnp.float32))
            n1c = jnp.sum((y1row == c).astype(jnp.float32))
            c22 = c22 + n2c * n2c
            c12 = c12 + n1c * n2c
        m1 = jnp.sum(acc_ref[0])
        t1 = jnp.sum(acc_ref[1])
        m2 = jnp.sum(acc_ref[2])
        t2 = jnp.sum(acc_ref[3])
        n1 = t1 - m1
        n2 = t2 - m2
        same1 = c22 - jnp.float32(_N2)
        different1 = jnp.float32(_N2) * jnp.float32(_N2) - c22
        same2 = c12
        different2 = jnp.float32(_N1) * jnp.float32(_N2) - c12
        out_ref[0, 0] = (m1 / same1 + m2 / same2) / (
            n1 / different1 + n2 / different2
        )


def kernel(y1, y2, f, f2):
    y1 = y1.astype(jnp.int32)
    y2 = y2.astype(jnp.int32)
    out = pl.pallas_call(
        _body,
        grid=(_NBLK,),
        in_specs=[
            pl.BlockSpec((1, _N2), lambda i: (0, 0)),
            pl.BlockSpec((1, _N1), lambda i: (0, 0)),
            pl.BlockSpec((_R, _N2), lambda i: (i, 0)),
            pl.BlockSpec((_R, _N2), lambda i: (i, 0)),
            pl.BlockSpec((_R, 1), lambda i: (i, 0)),
            pl.BlockSpec((_R, 1), lambda i: (i, 0)),
        ],
        out_specs=pl.BlockSpec(memory_space=pltpu.SMEM),
        out_shape=jax.ShapeDtypeStruct((1, 1), jnp.float32),
        scratch_shapes=[pltpu.VMEM((4, 8, _N2), jnp.float32)],
        compiler_params=pltpu.CompilerParams(
            dimension_semantics=("arbitrary",),
        ),
    )(
        y2.reshape(1, _N2),
        y1.reshape(1, _N1),
        f2,
        f,
        y2.reshape(_N2, 1),
        y1.reshape(_N1, 1),
    )
    return out[0, 0]


# speedup vs baseline: 1.0069x; 1.0069x over previous
import jax
import jax.numpy as jnp
from jax.experimental import pallas as pl
from jax.experimental.pallas import tpu as pltpu

_N1 = 8192
_N2 = 8192
_C = 16
_R = 256
_NBLK = _N1 // _R


def _body(y2r_ref, y1r_ref, f2_ref, f_ref, y2c_ref, y1c_ref, out_ref, acc_ref):
    i = pl.program_id(0)
    _CW = 512

    def chunked_sums(x_ref, rowlab):
        parts_m = []
        parts_t = []
        for c in range(_N2 // _CW):
            sl = slice(c * _CW, (c + 1) * _CW)
            xc = x_ref[:, sl]
            mc = rowlab == y2r_ref[:, sl]
            parts_m.append(jnp.sum(jnp.where(mc, xc, 0.0), axis=0, keepdims=True))
            parts_t.append(jnp.sum(xc, axis=0, keepdims=True))
        return (jnp.concatenate(parts_m, axis=1),
                jnp.concatenate(parts_t, axis=1))

    s_m1, s_t1 = chunked_sums(f2_ref, y2c_ref[...])
    s_m2, s_t2 = chunked_sums(f_ref, y1c_ref[...])

    @pl.when(i == 0)
    def _init():
        acc_ref[0] = s_m1
        acc_ref[1] = s_t1
        acc_ref[2] = s_m2
        acc_ref[3] = s_t2

    @pl.when(i > 0)
    def _accum():
        acc_ref[0] += s_m1
        acc_ref[1] += s_t1
        acc_ref[2] += s_m2
        acc_ref[3] += s_t2

    @pl.when(i == _NBLK - 1)
    def _finalize():
        y2row = y2r_ref[...]
        y1row = y1r_ref[...]
        c22 = jnp.float32(0.0)
        c12 = jnp.float32(0.0)
        for c in range(_C):
            n2c = jnp.sum((y2row == c).astype(jnp.float32))
            n1c = jnp.sum((y1row == c).astype(jnp.float32))
            c22 = c22 + n2c * n2c
            c12 = c12 + n1c * n2c
        m1 = jnp.sum(acc_ref[0])
        t1 = jnp.sum(acc_ref[1])
        m2 = jnp.sum(acc_ref[2])
        t2 = jnp.sum(acc_ref[3])
        n1 = t1 - m1
        n2 = t2 - m2
        same1 = c22 - jnp.float32(_N2)
        different1 = jnp.float32(_N2) * jnp.float32(_N2) - c22
        same2 = c12
        different2 = jnp.float32(_N1) * jnp.float32(_N2) - c12
        out_ref[0, 0] = (m1 / same1 + m2 / same2) / (
            n1 / different1 + n2 / different2
        )


def kernel(y1, y2, f, f2):
    y1 = y1.astype(jnp.int32)
    y2 = y2.astype(jnp.int32)
    out = pl.pallas_call(
        _body,
        grid=(_NBLK,),
        in_specs=[
            pl.BlockSpec((1, _N2), lambda i: (0, 0)),
            pl.BlockSpec((1, _N1), lambda i: (0, 0)),
            pl.BlockSpec((_R, _N2), lambda i: (i, 0)),
            pl.BlockSpec((_R, _N2), lambda i: (i, 0)),
            pl.BlockSpec((_R, 1), lambda i: (i, 0)),
            pl.BlockSpec((_R, 1), lambda i: (i, 0)),
        ],
        out_specs=pl.BlockSpec(memory_space=pltpu.SMEM),
        out_shape=jax.ShapeDtypeStruct((1, 1), jnp.float32),
        scratch_shapes=[pltpu.VMEM((4, 1, _N2), jnp.float32)],
        compiler_params=pltpu.CompilerParams(
            dimension_semantics=("arbitrary",),
        ),
    )(
        y2.reshape(1, _N2),
        y1.reshape(1, _N1),
        f2,
        f,
        y2.reshape(_N2, 1),
        y1.reshape(_N1, 1),
    )
    return out[0, 0]
